# Initial kernel scaffold; baseline (speedup 1.0000x reference)
import jax
import jax.numpy as jnp
from jax.experimental import pallas as pl


def kernel(h_fused, V, bucket_logits_q, bk, W_r, ln_gamma, ln_beta):
    raise NotImplementedError("write your pallas kernel here")


# speedup vs baseline: 3.0962x; 3.0962x over previous
import functools

import jax
import jax.numpy as jnp
from jax import lax
from jax.experimental import pallas as pl
from jax.experimental.pallas import tpu as pltpu
from jax.experimental.pallas import tpu_sc as plsc

N = 100000
D = 128
B = 64

CHUNK = 128
FULL_STEPS = N // CHUNK
TAIL = N - FULL_STEPS * CHUNK
NW = 32
STEPS_BASE = FULL_STEPS // NW
STEPS_REM = FULL_STEPS % NW

def _seg_body(v_hbm, bk_hbm, sums_out, cnts_out,
              idx_v, v_rows, idx_t, v_t, ones_v, stage, stage_c,
              sh_sums, sh_cnts):
    c = lax.axis_index("c")
    s = lax.axis_index("s")
    w = s * 2 + c

    def _fill_ones(i, carry):
        ones_v[i] = jnp.ones((16,), jnp.float32)
        return carry

    lax.fori_loop(0, CHUNK, _fill_ones, 0)

    def _fill_zeros(i, carry):
        for jj in range(D // 16):
            stage[i, pl.ds(jj * 16, 16)] = jnp.zeros((16,), jnp.float32)
        stage_c[i] = jnp.zeros((16,), jnp.float32)
        return carry

    lax.fori_loop(0, B, _fill_zeros, 0)

    @pl.when(s == 0)
    def _init():
        pltpu.sync_copy(stage, sh_sums)
        pltpu.sync_copy(stage_c, sh_cnts)

    plsc.subcore_barrier()

    n_steps = STEPS_BASE + jnp.where(w < STEPS_REM, 1, 0)

    def _step(j, carry):
        base = (w + j * NW) * CHUNK
        pltpu.sync_copy(bk_hbm.at[pl.ds(base, CHUNK)], idx_v)
        pltpu.sync_copy(v_hbm.at[pl.ds(base, CHUNK)], v_rows)
        pltpu.sync_copy(v_rows, sh_sums.at[idx_v], add=True)
        pltpu.sync_copy(ones_v, sh_cnts.at[idx_v], add=True)
        return carry

    lax.fori_loop(0, n_steps, _step, 0)

    @pl.when(w == 0)
    def _tail():
        base = FULL_STEPS * CHUNK
        pltpu.sync_copy(bk_hbm.at[pl.ds(base, TAIL)], idx_t)
        pltpu.sync_copy(v_hbm.at[pl.ds(base, TAIL)], v_t)
        pltpu.sync_copy(v_t, sh_sums.at[idx_t], add=True)
        pltpu.sync_copy(ones_v.at[pl.ds(0, TAIL)], sh_cnts.at[idx_t], add=True)

    plsc.subcore_barrier()

    @pl.when(s == 0)
    def _writeout():
        pltpu.sync_copy(sh_sums, stage)
        pltpu.sync_copy(sh_cnts, stage_c)

        @pl.when(c == 0)
        def _w0():
            pltpu.sync_copy(stage, sums_out.at[0])
            pltpu.sync_copy(stage_c, cnts_out.at[0])

        @pl.when(c == 1)
        def _w1():
            pltpu.sync_copy(stage, sums_out.at[1])
            pltpu.sync_copy(stage_c, cnts_out.at[1])


@functools.cache
def _seg_sums_sc():
    mesh = plsc.VectorSubcoreMesh(core_axis_name="c", subcore_axis_name="s")
    return pl.kernel(
        _seg_body,
        out_type=[
            jax.ShapeDtypeStruct((2, B, D), jnp.float32),
            jax.ShapeDtypeStruct((2, B, 16), jnp.float32),
        ],
        mesh=mesh,
        scratch_types=[
            pltpu.VMEM((CHUNK,), jnp.int32),
            pltpu.VMEM((CHUNK, D), jnp.float32),
            pltpu.VMEM((TAIL,), jnp.int32),
            pltpu.VMEM((TAIL, D), jnp.float32),
            pltpu.VMEM((CHUNK, 16), jnp.float32),
            pltpu.VMEM((B, D), jnp.float32),
            pltpu.VMEM((B, 16), jnp.float32),
            pltpu.VMEM_SHARED((B, D), jnp.float32),
            pltpu.VMEM_SHARED((B, 16), jnp.float32),
        ],
    )


ROWS_BLK = 2048
GRID = (N + ROWS_BLK - 1) // ROWS_BLK

_INV_LOG_B = 1.0 / float(jnp.log(jnp.float32(B)))


def _dense_body(h_ref, lg_ref, sums_ref, cnts_ref, wr_ref, g_ref, b_ref,
                out_ref, conf_ref):
    sums = sums_ref[0] + sums_ref[1]
    cnts = cnts_ref[0, :, 0:1] + cnts_ref[1, :, 0:1]
    gmean = jnp.sum(sums, axis=0, keepdims=True) * (1.0 / N)
    protos = sums / jnp.maximum(cnts, 1.0)
    protos = jnp.where(cnts == 0.0, gmean, protos)

    lg = lg_ref[...]
    m = jnp.max(lg, axis=-1, keepdims=True)
    e = jnp.exp(lg - m)
    p = e / jnp.sum(e, axis=-1, keepdims=True)
    ent = -jnp.sum(p * jnp.log(p + 1e-9), axis=-1, keepdims=True)
    gate = ent * _INV_LOG_B

    pw = jnp.dot(p, protos, preferred_element_type=jnp.float32)
    residual = lax.dot_general(pw, wr_ref[...], (((1,), (1,)), ((), ())),
                               preferred_element_type=jnp.float32)
    h = h_ref[...] + gate * residual
    mean = jnp.mean(h, axis=-1, keepdims=True)
    var = jnp.mean((h - mean) ** 2, axis=-1, keepdims=True)
    out_ref[...] = (h - mean) * lax.rsqrt(var + 1e-5) * g_ref[...] + b_ref[...]
    conf_ref[...] = 1.0 - gate


def _dense_stage(h_fused, bucket_logits_q, sums_p, cnts_p, W_r, g2, b2):
    out, conf2 = pl.pallas_call(
        _dense_body,
        grid=(GRID,),
        in_specs=[
            pl.BlockSpec((ROWS_BLK, D), lambda i: (i, 0)),
            pl.BlockSpec((ROWS_BLK, B), lambda i: (i, 0)),
            pl.BlockSpec((2, B, D), lambda i: (0, 0, 0)),
            pl.BlockSpec((2, B, 16), lambda i: (0, 0, 0)),
            pl.BlockSpec((D, D), lambda i: (0, 0)),
            pl.BlockSpec((1, D), lambda i: (0, 0)),
            pl.BlockSpec((1, D), lambda i: (0, 0)),
        ],
        out_specs=[
            pl.BlockSpec((ROWS_BLK, D), lambda i: (i, 0)),
            pl.BlockSpec((ROWS_BLK, 1), lambda i: (i, 0)),
        ],
        out_shape=[
            jax.ShapeDtypeStruct((N, D), jnp.float32),
            jax.ShapeDtypeStruct((N, 1), jnp.float32),
        ],
    )(h_fused, bucket_logits_q, sums_p, cnts_p, W_r, g2, b2)
    return out, conf2


def kernel(h_fused, V, bucket_logits_q, bk, W_r, ln_gamma, ln_beta):
    sums_p, cnts_p = _seg_sums_sc()(V, bk)
    out, conf2 = _dense_stage(
        h_fused, bucket_logits_q, sums_p, cnts_p, W_r,
        ln_gamma.reshape(1, D), ln_beta.reshape(1, D))
    return out, conf2.reshape(N)
